# Initial kernel scaffold; baseline (speedup 1.0000x reference)
import functools

import jax
import jax.numpy as jnp
from jax import lax
from jax.experimental import pallas as pl
from jax.experimental.pallas import tpu as pltpu
from jax.experimental.pallas import tpu_sc as plsc

B = 64
N = 32768
K_RANK = int(N * 0.8)
L = 16
NVEC = N // L
NC = 2
NS = 16
ROWS_PER_W = B // (NC * NS)


def _build():
    mesh = plsc.VectorSubcoreMesh(core_axis_name="c", subcore_axis_name="s")

    @functools.partial(
        pl.kernel,
        mesh=mesh,
        out_type=jax.ShapeDtypeStruct((B, N), jnp.float32),
        scratch_types=[
            pltpu.VMEM((N,), jnp.float32),
            pltpu.VMEM((256 * L,), jnp.int32),
        ],
    )
    def sc_kernel(x_hbm, out_hbm, xbuf, hist):
        wid = lax.axis_index("s") * NC + lax.axis_index("c")
        lanes = lax.iota(jnp.int32, L)
        ones = jnp.ones((L,), jnp.int32)
        zeros16 = jnp.zeros((L,), jnp.int32)

        def keys_of(xv):
            b = plsc.bitcast(xv, jnp.int32)
            m = lax.shift_right_arithmetic(b, 31) | jnp.int32(-(2 ** 31))
            return plsc.bitcast(b ^ m, jnp.uint32)

        def do_row(r, _):
            row = wid * ROWS_PER_W + r
            pltpu.sync_copy(x_hbm.at[row], xbuf)

            prefix = jnp.uint32(0)
            k_rem = jnp.int32(K_RANK)

            for p in range(4):
                shift = 24 - 8 * p

                def zbody(i, c):
                    hist[pl.ds(i * L, L)] = zeros16
                    return c

                lax.fori_loop(0, 256, zbody, 0)

                def sbody(i, c, _shift=shift, _p=p, _prefix=prefix):
                    xv = xbuf[pl.ds(i * L, L)]
                    key = keys_of(xv)
                    digit = (key >> jnp.uint32(_shift)) & jnp.uint32(0xFF)
                    idx = digit.astype(jnp.int32) * L + lanes
                    if _p == 0:
                        plsc.addupdate_scatter(hist, [idx], ones)
                    else:
                        pm = (key >> jnp.uint32(_shift + 8)) == _prefix
                        plsc.addupdate_scatter(hist, [idx], ones, mask=pm)
                    return c

                lax.fori_loop(0, NVEC, sbody, 0)

                def chunk_body(c, carry):
                    found, dstar, below, run = carry

                    def acc_body(j, acc):
                        return acc + hist[pl.ds((c * 16 + j) * L, L)]

                    Cc = lax.fori_loop(0, 16, acc_body, zeros16)
                    tot = jnp.sum(Cc)
                    hit = jnp.logical_and(jnp.logical_not(found),
                                          run + tot > k_rem)

                    def in_search(_):
                        def dig_body(j, dc):
                            f2, d2, b2, r2 = dc
                            s = jnp.sum(hist[pl.ds((c * 16 + j) * L, L)])
                            h2 = jnp.logical_and(jnp.logical_not(f2),
                                                 r2 + s > k_rem)
                            d2 = jnp.where(h2, c * 16 + j, d2)
                            b2 = jnp.where(h2, r2, b2)
                            f2 = jnp.logical_or(f2, h2)
                            return (f2, d2, b2, r2 + s)

                        f2, d2, b2, _ = lax.fori_loop(
                            0, 16, dig_body,
                            (jnp.bool_(False), dstar, below, run))
                        return (d2, b2)

                    def no_search(_):
                        return (dstar, below)

                    dstar2, below2 = lax.cond(hit, in_search, no_search, 0)
                    return (jnp.logical_or(found, hit), dstar2, below2,
                            run + tot)

                found, dstar, below, run = lax.fori_loop(
                    0, 16, chunk_body,
                    (jnp.bool_(False), jnp.int32(0), jnp.int32(0),
                     jnp.int32(0)))

                k_rem = k_rem - below
                prefix = (prefix << jnp.uint32(8)) | dstar.astype(jnp.uint32)

            neg = (prefix & jnp.uint32(0x80000000)) == jnp.uint32(0)
            bits = jnp.where(neg, ~prefix, prefix ^ jnp.uint32(0x80000000))
            cutoff_v = plsc.bitcast(jnp.broadcast_to(bits, (L,)), jnp.float32)

            def obody(i, c):
                xv = xbuf[pl.ds(i * L, L)]
                xbuf[pl.ds(i * L, L)] = jnp.maximum(xv - cutoff_v, 0.0)
                return c

            lax.fori_loop(0, NVEC, obody, 0)
            pltpu.sync_copy(xbuf, out_hbm.at[row])
            return 0

        lax.fori_loop(0, ROWS_PER_W, do_row, 0)

    return sc_kernel


_sc_kernel = _build()


def kernel(input):
    return _sc_kernel(input)


# speedup vs baseline: 6.2568x; 6.2568x over previous
import functools

import jax
import jax.numpy as jnp
from jax import lax
from jax.experimental import pallas as pl
from jax.experimental.pallas import tpu as pltpu
from jax.experimental.pallas import tpu_sc as plsc

B = 64
N = 32768
K_RANK = int(N * 0.8)
L = 16
NVEC = N // L
NC = 2
NS = 16
ROWS_PER_W = B // (NC * NS)


def _build():
    mesh = plsc.VectorSubcoreMesh(core_axis_name="c", subcore_axis_name="s")

    @functools.partial(
        pl.kernel,
        mesh=mesh,
        out_type=jax.ShapeDtypeStruct((B, N), jnp.float32),
        scratch_types=[
            pltpu.VMEM((N,), jnp.float32),
            pltpu.VMEM((256 * L,), jnp.int32),
        ],
        compiler_params=pltpu.CompilerParams(needs_layout_passes=False),
    )
    def sc_kernel(x_hbm, out_hbm, xbuf, hist):
        wid = lax.axis_index("s") * NC + lax.axis_index("c")
        lanes = lax.iota(jnp.int32, L)
        ones = jnp.ones((L,), jnp.int32)
        zeros16 = jnp.zeros((L,), jnp.int32)

        def keys_of(xv):
            b = lax.bitcast_convert_type(xv, jnp.int32)
            m = lax.shift_right_arithmetic(b, 31) | jnp.int32(-(2 ** 31))
            return lax.bitcast_convert_type(b ^ m, jnp.uint32)

        def do_row(r, _):
            row = wid * ROWS_PER_W + r
            pltpu.sync_copy(x_hbm.at[row], xbuf)

            prefix = jnp.uint32(0)
            k_rem = jnp.int32(K_RANK)

            for p in range(4):
                shift = 24 - 8 * p

                def zbody(i, c):
                    hist[pl.ds(i * L, L)] = zeros16
                    return c

                lax.fori_loop(0, 256, zbody, 0)

                def sbody(i, c, _shift=shift, _p=p, _prefix=prefix):
                    xv = xbuf[pl.ds(i * L, L)]
                    key = keys_of(xv)
                    digit = (key >> jnp.uint32(_shift)) & jnp.uint32(0xFF)
                    idx = digit.astype(jnp.int32) * L + lanes
                    if _p == 0:
                        plsc.addupdate_scatter(hist, [idx], ones)
                    else:
                        pm = (key >> jnp.uint32(_shift + 8)) == _prefix
                        plsc.addupdate_scatter(hist, [idx], ones, mask=pm)
                    return c

                lax.fori_loop(0, NVEC, sbody, 0)

                def chunk_body(c, carry):
                    found, dstar, below, run = carry

                    def acc_body(j, acc):
                        return acc + hist[pl.ds((c * 16 + j) * L, L)]

                    Cc = lax.fori_loop(0, 16, acc_body, zeros16)
                    tot = jnp.sum(Cc)
                    hit = jnp.logical_and(jnp.logical_not(found),
                                          run + tot > k_rem)

                    def in_search(_):
                        def dig_body(j, dc):
                            f2, d2, b2, r2 = dc
                            s = jnp.sum(hist[pl.ds((c * 16 + j) * L, L)])
                            h2 = jnp.logical_and(jnp.logical_not(f2),
                                                 r2 + s > k_rem)
                            d2 = jnp.where(h2, c * 16 + j, d2)
                            b2 = jnp.where(h2, r2, b2)
                            f2 = jnp.logical_or(f2, h2)
                            return (f2, d2, b2, r2 + s)

                        f2, d2, b2, _ = lax.fori_loop(
                            0, 16, dig_body,
                            (jnp.bool_(False), dstar, below, run))
                        return (d2, b2)

                    def no_search(_):
                        return (dstar, below)

                    dstar2, below2 = lax.cond(hit, in_search, no_search, 0)
                    return (jnp.logical_or(found, hit), dstar2, below2,
                            run + tot)

                found, dstar, below, run = lax.fori_loop(
                    0, 16, chunk_body,
                    (jnp.bool_(False), jnp.int32(0), jnp.int32(0),
                     jnp.int32(0)))

                k_rem = k_rem - below
                prefix = (prefix << jnp.uint32(8)) | dstar.astype(jnp.uint32)

            neg = (prefix & jnp.uint32(0x80000000)) == jnp.uint32(0)
            bits = jnp.where(neg, ~prefix, prefix ^ jnp.uint32(0x80000000))
            cutoff_v = lax.bitcast_convert_type(
                jnp.broadcast_to(bits, (L,)), jnp.float32)

            def obody(i, c):
                xv = xbuf[pl.ds(i * L, L)]
                xbuf[pl.ds(i * L, L)] = jnp.maximum(xv - cutoff_v, 0.0)
                return c

            lax.fori_loop(0, NVEC, obody, 0)
            pltpu.sync_copy(xbuf, out_hbm.at[row])
            return 0

        lax.fori_loop(0, ROWS_PER_W, do_row, 0)

    return sc_kernel


_sc_kernel = _build()


def kernel(input):
    return _sc_kernel(input)
